# baseline (speedup 1.0000x reference)
import functools

import jax
import jax.numpy as jnp
from jax import lax
from jax.experimental import pallas as pl
from jax.experimental.pallas import tpu as pltpu
from jax.experimental.pallas import tpu_sc as plsc

N = 10000
E = 320000
D = 128
H = 128
C = 8
G = 64

NC = 2
NS = 16
NW = NC * NS
CHUNK = 128
NCHUNK = 80
NSTAGE = 2
STAGE = NCHUNK // NSTAGE
NBUF = 2
EPT = NCHUNK * CHUNK
EPAD = NW * EPT
R0 = 624
R_LAST = N - R0 * (NS - 1)
SC_ROWS = N + 976
N_TRASH = SC_ROWS - N

BLK = 1000
GRID = N // BLK

_f32 = jnp.float32
_HI = lax.Precision.HIGHEST

_sc_mesh = plsc.VectorSubcoreMesh(core_axis_name="c", subcore_axis_name="s")


def _per_tile_rows(s, fn):
    @pl.when(s < NS - 1)
    def _():
        fn(s * R0, R0)

    @pl.when(s == NS - 1)
    def _():
        fn(N - R_LAST, R_LAST)


def _hist_body(dst_hbm, ones_hbm, zeros_hbm, out_hbm, dst_v, ones_v, tbl_sh):
    c = lax.axis_index("c")
    s = lax.axis_index("s")
    gwid = c * NS + s
    _per_tile_rows(s, lambda off, nr: pltpu.sync_copy(
        zeros_hbm.at[pl.ds(0, nr)], tbl_sh.at[pl.ds(off, nr)]))
    pltpu.sync_copy(dst_hbm.at[gwid], dst_v)
    pltpu.sync_copy(ones_hbm, ones_v)
    plsc.subcore_barrier()

    def body(j, carry):
        pltpu.sync_copy(ones_v, tbl_sh.at[dst_v.at[j]], add=True)
        return carry

    lax.fori_loop(0, NCHUNK, body, None)
    plsc.subcore_barrier()
    _per_tile_rows(s, lambda off, nr: pltpu.sync_copy(
        tbl_sh.at[pl.ds(off, nr)], out_hbm.at[c, pl.ds(off, nr)]))


_hist = pl.kernel(
    _hist_body,
    out_type=jax.ShapeDtypeStruct((NC, N, 16), _f32),
    mesh=_sc_mesh,
    scratch_types=[
        pltpu.VMEM((NCHUNK, CHUNK), jnp.int32),
        pltpu.VMEM((CHUNK, 16), _f32),
        pltpu.VMEM_SHARED((SC_ROWS, 16), _f32),
    ],
    compiler_params=pltpu.CompilerParams(use_tc_tiling_on_sc=False),
)


def _agg_body(h_hbm, src_hbm, dst_hbm, out_hbm, src_v, dst_v, buf, acc_sh,
              gsems, ssems):
    c = lax.axis_index("c")
    s = lax.axis_index("s")
    gwid = c * NS + s
    _per_tile_rows(s, lambda off, nr: pltpu.sync_copy(
        h_hbm.at[pl.ds(off, nr)], acc_sh.at[pl.ds(off, nr)]))
    plsc.subcore_barrier()

    for stg in range(NSTAGE):
        pltpu.sync_copy(src_hbm.at[gwid, pl.ds(stg * STAGE, STAGE)], src_v)
        pltpu.sync_copy(dst_hbm.at[gwid, pl.ds(stg * STAGE, STAGE)], dst_v)
        for b in range(NBUF):
            pltpu.async_copy(h_hbm.at[src_v.at[b]], buf.at[b], gsems.at[b])

        def body(r, carry):
            for b in range(NBUF):
                j = r * NBUF + b
                pltpu.make_async_copy(h_hbm.at[src_v.at[j]], buf.at[b],
                                      gsems.at[b]).wait()
                pltpu.async_copy(buf.at[b], acc_sh.at[dst_v.at[j]],
                                 ssems.at[b], add=True)
            for b in range(NBUF):
                j = r * NBUF + b
                pltpu.make_async_copy(buf.at[b], acc_sh.at[dst_v.at[j]],
                                      ssems.at[b]).wait()

                @pl.when(j + NBUF < STAGE)
                def _(j=j, b=b):
                    pltpu.async_copy(h_hbm.at[src_v.at[j + NBUF]], buf.at[b],
                                     gsems.at[b])
            return carry

        lax.fori_loop(0, STAGE // NBUF, body, None)
    plsc.subcore_barrier()
    _per_tile_rows(s, lambda off, nr: pltpu.sync_copy(
        acc_sh.at[pl.ds(off, nr)], out_hbm.at[c, pl.ds(off, nr)]))


_agg = pl.kernel(
    _agg_body,
    out_type=jax.ShapeDtypeStruct((NC, N, D), _f32),
    mesh=_sc_mesh,
    scratch_types=[
        pltpu.VMEM((STAGE, CHUNK), jnp.int32),
        pltpu.VMEM((STAGE, CHUNK), jnp.int32),
        pltpu.VMEM((NBUF, CHUNK, D), _f32),
        pltpu.VMEM_SHARED((SC_ROWS, D), _f32),
        pltpu.SemaphoreType.DMA((NBUF,)),
        pltpu.SemaphoreType.DMA((NBUF,)),
    ],
)


def _dinv(da_ref, db_ref):
    deg = 1.0 + da_ref[:, :1] + db_ref[:, :1]
    return lax.rsqrt(deg)


def _tc1_body(x_ref, w_ref, da_ref, db_ref, o_ref):
    xw = jnp.dot(x_ref[...], w_ref[...], preferred_element_type=_f32,
                 precision=_HI)
    o_ref[...] = _dinv(da_ref, db_ref) * xw


_row_spec = pl.BlockSpec((BLK, D), lambda i: (i, 0))
_deg_spec = pl.BlockSpec((BLK, 16), lambda i: (i, 0))


def _const_spec(shape):
    return pl.BlockSpec(shape, lambda i: tuple(0 for _ in shape))


_tc1 = pl.pallas_call(
    _tc1_body,
    grid=(GRID,),
    in_specs=[_row_spec, _const_spec((D, H)), _deg_spec, _deg_spec],
    out_specs=_row_spec,
    out_shape=jax.ShapeDtypeStruct((N, H), _f32),
)


def _tc2_body(p0_ref, p1_ref, hp_ref, da_ref, db_ref, w_ref, b_ref, o_ref):
    dinv = _dinv(da_ref, db_ref)
    acc = p0_ref[...] + p1_ref[...] - hp_ref[...]
    h1 = jnp.maximum(dinv * acc + b_ref[...], 0.0)
    o_ref[...] = dinv * jnp.dot(h1, w_ref[...], preferred_element_type=_f32,
                                precision=_HI)


_tc2 = pl.pallas_call(
    _tc2_body,
    grid=(GRID,),
    in_specs=[_row_spec, _row_spec, _row_spec, _deg_spec, _deg_spec,
              _const_spec((H, H)), _const_spec((1, H))],
    out_specs=_row_spec,
    out_shape=jax.ShapeDtypeStruct((N, H), _f32),
)


def _tc3_body(q0_ref, q1_ref, hp_ref, da_ref, db_ref, bt_ref, b2_ref,
              l1w_ref, l1b_ref, l2w_ref, l2b_ref, l3w_ref, l3b_ref,
              o_ref, gmax_s, gsum_s, cnt_s):
    i = pl.program_id(0)

    @pl.when(i == 0)
    def _():
        gmax_s[...] = jnp.full((G, H), -jnp.inf, _f32)
        gsum_s[...] = jnp.zeros((G, H), _f32)
        cnt_s[...] = jnp.zeros((G, H), _f32)

    dinv = _dinv(da_ref, db_ref)
    acc = q0_ref[...] + q1_ref[...] - hp_ref[...]
    h2 = jnp.maximum(dinv * acc + b2_ref[...], 0.0)

    bids = bt_ref[...]
    gids = lax.broadcasted_iota(jnp.int32, (BLK, G), 1)
    mask = (bids == gids).astype(_f32)
    dims = (((0,), (0,)), ((), ()))
    gsum_s[...] += lax.dot_general(mask, h2, dims, precision=_HI)
    cnt_s[...] += lax.dot_general(mask, jnp.ones_like(h2), dims, precision=_HI)

    lo = jnp.min(bids)
    hi = jnp.max(bids)
    for g in range(G):
        @pl.when((lo <= g) & (g <= hi))
        def _(g=g):
            m = jnp.where(bids == g, h2, -jnp.inf)
            gmax_s[pl.ds(g, 1), :] = jnp.maximum(
                gmax_s[pl.ds(g, 1), :], jnp.max(m, axis=0, keepdims=True))

    @pl.when(i == pl.num_programs(0) - 1)
    def _():
        cnt = cnt_s[:, :1]
        gmean = gsum_s[...] / jnp.maximum(cnt, 1.0)
        gmax = jnp.where(cnt > 0.0, gmax_s[...], 0.0)
        l1w = l1w_ref[...]
        r = jnp.maximum(
            jnp.dot(gmax, l1w[:H, :], preferred_element_type=_f32, precision=_HI)
            + jnp.dot(gmean, l1w[H:, :], preferred_element_type=_f32, precision=_HI)
            + l1b_ref[...], 0.0)
        r = jnp.maximum(
            jnp.dot(r, l2w_ref[...], preferred_element_type=_f32, precision=_HI)
            + l2b_ref[...], 0.0)
        logits = jnp.dot(r, l3w_ref[...], preferred_element_type=_f32,
                         precision=_HI) + l3b_ref[...]
        mx = jnp.max(logits, axis=-1, keepdims=True)
        z = logits - mx
        o_ref[...] = z - jnp.log(jnp.sum(jnp.exp(z), axis=-1, keepdims=True))


_tc3 = pl.pallas_call(
    _tc3_body,
    grid=(GRID,),
    in_specs=[_row_spec, _row_spec, _row_spec, _deg_spec, _deg_spec,
              pl.BlockSpec((BLK, 1), lambda i: (i, 0)),
              _const_spec((1, H)),
              _const_spec((2 * H, H)), _const_spec((1, H)),
              _const_spec((H, H // 2)), _const_spec((1, H // 2)),
              _const_spec((H // 2, C)), _const_spec((1, C))],
    out_specs=pl.BlockSpec((G, C), lambda i: (0, 0)),
    out_shape=jax.ShapeDtypeStruct((G, C), _f32),
    scratch_shapes=[pltpu.VMEM((G, H), _f32), pltpu.VMEM((G, H), _f32),
                    pltpu.VMEM((G, H), _f32)],
)


def kernel(x, edge_index, batch, W1, b1, W2, b2, lin1_W, lin1_b,
           lin2_W, lin2_b, lin3_W, lin3_b):
    pad = EPAD - E
    ppt = pad // NW
    ept_real = E // NW
    pad_src = (jnp.arange(pad, dtype=jnp.int32) * 131) % N
    pad_dst = N + (jnp.arange(pad, dtype=jnp.int32) % N_TRASH)
    src_p = jnp.concatenate(
        [edge_index[0].reshape(NW, ept_real), pad_src.reshape(NW, ppt)],
        axis=1).reshape(NW, NCHUNK, CHUNK)
    dst_p = jnp.concatenate(
        [edge_index[1].reshape(NW, ept_real), pad_dst.reshape(NW, ppt)],
        axis=1).reshape(NW, NCHUNK, CHUNK)
    ones = jnp.ones((CHUNK, 16), _f32)
    zeros = jnp.zeros((R_LAST, 16), _f32)

    hist = _hist(dst_p, ones, zeros)
    degA = hist[0]
    degB = hist[1]

    h1p = _tc1(x, W1, degA, degB)
    parts1 = _agg(h1p, src_p, dst_p)
    h2p = _tc2(parts1[0], parts1[1], h1p, degA, degB,
               W2, b1.reshape(1, H))
    parts2 = _agg(h2p, src_p, dst_p)
    out = _tc3(parts2[0], parts2[1], h2p, degA, degB,
               batch.reshape(N, 1), b2.reshape(1, H),
               lin1_W, lin1_b.reshape(1, H),
               lin2_W, lin2_b.reshape(1, H // 2),
               lin3_W, lin3_b.reshape(1, C))
    return out


# speedup vs baseline: 1.2027x; 1.2027x over previous
import functools

import jax
import jax.numpy as jnp
from jax import lax
from jax.experimental import pallas as pl
from jax.experimental.pallas import tpu as pltpu
from jax.experimental.pallas import tpu_sc as plsc

N = 10000
E = 320000
D = 128
H = 128
C = 8
G = 64

NC = 2
NS = 16
NW = NC * NS
CHUNK = 128
NCHUNK = 80
NSTAGE = 2
STAGE = NCHUNK // NSTAGE
NBUF = 2
EPT = NCHUNK * CHUNK
EPAD = NW * EPT
R0 = 624
R_LAST = N - R0 * (NS - 1)
SC_ROWS = N + 976
N_TRASH = SC_ROWS - N

BLK = 1000
GRID = N // BLK

_f32 = jnp.float32
_HI = lax.Precision.HIGHEST

_sc_mesh = plsc.VectorSubcoreMesh(core_axis_name="c", subcore_axis_name="s")


def _per_tile_rows(s, fn):
    @pl.when(s < NS - 1)
    def _():
        fn(s * R0, R0)

    @pl.when(s == NS - 1)
    def _():
        fn(N - R_LAST, R_LAST)


def _hist_body(dst_hbm, ones_hbm, zeros_hbm, out_hbm, dst_v, ones_v, tbl_sh):
    c = lax.axis_index("c")
    s = lax.axis_index("s")
    gwid = c * NS + s
    _per_tile_rows(s, lambda off, nr: pltpu.sync_copy(
        zeros_hbm.at[pl.ds(0, nr)], tbl_sh.at[pl.ds(off, nr)]))
    pltpu.sync_copy(dst_hbm.at[gwid], dst_v)
    pltpu.sync_copy(ones_hbm, ones_v)
    plsc.subcore_barrier()

    def body(j, carry):
        pltpu.sync_copy(ones_v, tbl_sh.at[dst_v.at[j]], add=True)
        return carry

    lax.fori_loop(0, NCHUNK, body, None)
    plsc.subcore_barrier()
    _per_tile_rows(s, lambda off, nr: pltpu.sync_copy(
        tbl_sh.at[pl.ds(off, nr)], out_hbm.at[c, pl.ds(off, nr)]))


_hist = pl.kernel(
    _hist_body,
    out_type=jax.ShapeDtypeStruct((NC, N, 16), _f32),
    mesh=_sc_mesh,
    scratch_types=[
        pltpu.VMEM((NCHUNK, CHUNK), jnp.int32),
        pltpu.VMEM((CHUNK, 16), _f32),
        pltpu.VMEM_SHARED((SC_ROWS, 16), _f32),
    ],
    compiler_params=pltpu.CompilerParams(use_tc_tiling_on_sc=False),
)


def _agg_body(h_hbm, src_hbm, dst_hbm, out_hbm, src_v, dst_v, buf, acc_sh,
              gsems):
    c = lax.axis_index("c")
    s = lax.axis_index("s")
    gwid = c * NS + s
    _per_tile_rows(s, lambda off, nr: pltpu.sync_copy(
        h_hbm.at[pl.ds(off, nr)], acc_sh.at[pl.ds(off, nr)]))
    plsc.subcore_barrier()

    for stg in range(NSTAGE):
        pltpu.sync_copy(src_hbm.at[gwid, pl.ds(stg * STAGE, STAGE)], src_v)
        pltpu.sync_copy(dst_hbm.at[gwid, pl.ds(stg * STAGE, STAGE)], dst_v)
        for b in range(NBUF):
            pltpu.async_copy(h_hbm.at[src_v.at[b]], buf.at[b], gsems.at[b])

        def body(r, carry):
            for b in range(NBUF):
                j = r * NBUF + b
                pltpu.make_async_copy(h_hbm.at[src_v.at[j]], buf.at[b],
                                      gsems.at[b]).wait()
                pltpu.sync_copy(buf.at[b], acc_sh.at[dst_v.at[j]], add=True)

                @pl.when(j + NBUF < STAGE)
                def _(j=j, b=b):
                    pltpu.async_copy(h_hbm.at[src_v.at[j + NBUF]], buf.at[b],
                                     gsems.at[b])
            return carry

        lax.fori_loop(0, STAGE // NBUF, body, None)
    plsc.subcore_barrier()
    _per_tile_rows(s, lambda off, nr: pltpu.sync_copy(
        acc_sh.at[pl.ds(off, nr)], out_hbm.at[c, pl.ds(off, nr)]))


_agg = pl.kernel(
    _agg_body,
    out_type=jax.ShapeDtypeStruct((NC, N, D), _f32),
    mesh=_sc_mesh,
    scratch_types=[
        pltpu.VMEM((STAGE, CHUNK), jnp.int32),
        pltpu.VMEM((STAGE, CHUNK), jnp.int32),
        pltpu.VMEM((NBUF, CHUNK, D), _f32),
        pltpu.VMEM_SHARED((SC_ROWS, D), _f32),
        pltpu.SemaphoreType.DMA((NBUF,)),
    ],
)


def _dinv(da_ref, db_ref):
    deg = 1.0 + da_ref[:, :1] + db_ref[:, :1]
    return lax.rsqrt(deg)


def _tc1_body(x_ref, w_ref, da_ref, db_ref, o_ref):
    xw = jnp.dot(x_ref[...], w_ref[...], preferred_element_type=_f32,
                 precision=_HI)
    o_ref[...] = _dinv(da_ref, db_ref) * xw


_row_spec = pl.BlockSpec((BLK, D), lambda i: (i, 0))
_deg_spec = pl.BlockSpec((BLK, 16), lambda i: (i, 0))


def _const_spec(shape):
    return pl.BlockSpec(shape, lambda i: tuple(0 for _ in shape))


_tc1 = pl.pallas_call(
    _tc1_body,
    grid=(GRID,),
    in_specs=[_row_spec, _const_spec((D, H)), _deg_spec, _deg_spec],
    out_specs=_row_spec,
    out_shape=jax.ShapeDtypeStruct((N, H), _f32),
)


def _tc2_body(p0_ref, p1_ref, hp_ref, da_ref, db_ref, w_ref, b_ref, o_ref):
    dinv = _dinv(da_ref, db_ref)
    acc = p0_ref[...] + p1_ref[...] - hp_ref[...]
    h1 = jnp.maximum(dinv * acc + b_ref[...], 0.0)
    o_ref[...] = dinv * jnp.dot(h1, w_ref[...], preferred_element_type=_f32,
                                precision=_HI)


_tc2 = pl.pallas_call(
    _tc2_body,
    grid=(GRID,),
    in_specs=[_row_spec, _row_spec, _row_spec, _deg_spec, _deg_spec,
              _const_spec((H, H)), _const_spec((1, H))],
    out_specs=_row_spec,
    out_shape=jax.ShapeDtypeStruct((N, H), _f32),
)


def _tc3_body(q0_ref, q1_ref, hp_ref, da_ref, db_ref, bt_ref, b2_ref,
              l1w_ref, l1b_ref, l2w_ref, l2b_ref, l3w_ref, l3b_ref,
              o_ref, gmax_s, gsum_s, cnt_s):
    i = pl.program_id(0)

    @pl.when(i == 0)
    def _():
        gmax_s[...] = jnp.full((G, H), -jnp.inf, _f32)
        gsum_s[...] = jnp.zeros((G, H), _f32)
        cnt_s[...] = jnp.zeros((G, H), _f32)

    dinv = _dinv(da_ref, db_ref)
    acc = q0_ref[...] + q1_ref[...] - hp_ref[...]
    h2 = jnp.maximum(dinv * acc + b2_ref[...], 0.0)

    bids = bt_ref[...]
    gids = lax.broadcasted_iota(jnp.int32, (BLK, G), 1)
    mask = (bids == gids).astype(_f32)
    dims = (((0,), (0,)), ((), ()))
    gsum_s[...] += lax.dot_general(mask, h2, dims, precision=_HI)
    cnt_s[...] += lax.dot_general(mask, jnp.ones_like(h2), dims, precision=_HI)

    lo = jnp.min(bids)
    hi = jnp.max(bids)
    for g in range(G):
        @pl.when((lo <= g) & (g <= hi))
        def _(g=g):
            m = jnp.where(bids == g, h2, -jnp.inf)
            gmax_s[pl.ds(g, 1), :] = jnp.maximum(
                gmax_s[pl.ds(g, 1), :], jnp.max(m, axis=0, keepdims=True))

    @pl.when(i == pl.num_programs(0) - 1)
    def _():
        cnt = cnt_s[:, :1]
        gmean = gsum_s[...] / jnp.maximum(cnt, 1.0)
        gmax = jnp.where(cnt > 0.0, gmax_s[...], 0.0)
        l1w = l1w_ref[...]
        r = jnp.maximum(
            jnp.dot(gmax, l1w[:H, :], preferred_element_type=_f32, precision=_HI)
            + jnp.dot(gmean, l1w[H:, :], preferred_element_type=_f32, precision=_HI)
            + l1b_ref[...], 0.0)
        r = jnp.maximum(
            jnp.dot(r, l2w_ref[...], preferred_element_type=_f32, precision=_HI)
            + l2b_ref[...], 0.0)
        logits = jnp.dot(r, l3w_ref[...], preferred_element_type=_f32,
                         precision=_HI) + l3b_ref[...]
        mx = jnp.max(logits, axis=-1, keepdims=True)
        z = logits - mx
        o_ref[...] = z - jnp.log(jnp.sum(jnp.exp(z), axis=-1, keepdims=True))


_tc3 = pl.pallas_call(
    _tc3_body,
    grid=(GRID,),
    in_specs=[_row_spec, _row_spec, _row_spec, _deg_spec, _deg_spec,
              pl.BlockSpec((BLK, 1), lambda i: (i, 0)),
              _const_spec((1, H)),
              _const_spec((2 * H, H)), _const_spec((1, H)),
              _const_spec((H, H // 2)), _const_spec((1, H // 2)),
              _const_spec((H // 2, C)), _const_spec((1, C))],
    out_specs=pl.BlockSpec((G, C), lambda i: (0, 0)),
    out_shape=jax.ShapeDtypeStruct((G, C), _f32),
    scratch_shapes=[pltpu.VMEM((G, H), _f32), pltpu.VMEM((G, H), _f32),
                    pltpu.VMEM((G, H), _f32)],
)


def kernel(x, edge_index, batch, W1, b1, W2, b2, lin1_W, lin1_b,
           lin2_W, lin2_b, lin3_W, lin3_b):
    pad = EPAD - E
    ppt = pad // NW
    ept_real = E // NW
    pad_src = (jnp.arange(pad, dtype=jnp.int32) * 131) % N
    pad_dst = N + (jnp.arange(pad, dtype=jnp.int32) % N_TRASH)
    src_p = jnp.concatenate(
        [edge_index[0].reshape(NW, ept_real), pad_src.reshape(NW, ppt)],
        axis=1).reshape(NW, NCHUNK, CHUNK)
    dst_p = jnp.concatenate(
        [edge_index[1].reshape(NW, ept_real), pad_dst.reshape(NW, ppt)],
        axis=1).reshape(NW, NCHUNK, CHUNK)
    ones = jnp.ones((CHUNK, 16), _f32)
    zeros = jnp.zeros((R_LAST, 16), _f32)

    hist = _hist(dst_p, ones, zeros)
    degA = hist[0]
    degB = hist[1]

    h1p = _tc1(x, W1, degA, degB)
    parts1 = _agg(h1p, src_p, dst_p)
    h2p = _tc2(parts1[0], parts1[1], h1p, degA, degB,
               W2, b1.reshape(1, H))
    parts2 = _agg(h2p, src_p, dst_p)
    out = _tc3(parts2[0], parts2[1], h2p, degA, degB,
               batch.reshape(N, 1), b2.reshape(1, H),
               lin1_W, lin1_b.reshape(1, H),
               lin2_W, lin2_b.reshape(1, H // 2),
               lin3_W, lin3_b.reshape(1, C))
    return out
